# baseline (speedup 1.0000x reference)
import functools

import jax
import jax.numpy as jnp
from jax import lax
from jax.experimental import pallas as pl
from jax.experimental.pallas import tpu as pltpu
from jax.experimental.pallas import tpu_sc as plsc

N_NODES = 10000
RANK = 128

NC = 2
NS = 16
NW = NC * NS

CHUNK = 88
N_PAD = 10240
ROWS_PER_TILE = N_PAD // NS
ZB = 80
REL_PAD = 256


def _sc_accumulate(h, rel_p, srcx, dstx, etx, cpt):
    mesh = plsc.VectorSubcoreMesh(core_axis_name="c", subcore_axis_name="s")

    @functools.partial(
        pl.kernel,
        mesh=mesh,
        out_type=jax.ShapeDtypeStruct((NC * N_PAD, RANK), jnp.float32),
        scratch_types=[
            pltpu.VMEM_SHARED((N_PAD, RANK), jnp.float32),
            pltpu.VMEM_SHARED((REL_PAD, RANK), jnp.float32),
            [pltpu.VMEM((CHUNK,), jnp.int32)] * 2,
            [pltpu.VMEM((CHUNK,), jnp.int32)] * 2,
            [pltpu.VMEM((CHUNK,), jnp.int32)] * 2,
            pltpu.VMEM((2, CHUNK, RANK), jnp.float32),
            pltpu.VMEM((2, CHUNK, RANK), jnp.float32),
            [pltpu.SemaphoreType.DMA] * 2,
            [pltpu.SemaphoreType.DMA] * 2,
            [pltpu.SemaphoreType.DMA] * 2,
            [pltpu.SemaphoreType.DMA] * 2,
        ],
    )
    def body(h_hbm, rel_hbm, srcx_hbm, dstx_hbm, etx_hbm, acc_hbm,
             acc_sh, rel_sh, ci, dx, ei, hrows, rrows, gh, gr, sh, sr):
        c = lax.axis_index("c")
        s = lax.axis_index("s")
        wid = s * NC + c

        def mk_gather_h(b):
            return pltpu.make_async_copy(h_hbm.at[ci[b]], hrows.at[b], gh[b])

        def mk_gather_r(b):
            return pltpu.make_async_copy(rel_sh.at[ei[b]], rrows.at[b], gr[b])

        def start_scat(b):
            pltpu.async_copy(hrows.at[b], acc_sh.at[dx[b]], sh[b], add=True)
            pltpu.async_copy(rrows.at[b], acc_sh.at[dx[b]], sr[b], add=True)

        def drain_scat(b):
            pltpu.make_async_copy(hrows.at[b], acc_sh.at[dx[b]], sh[b]).wait()
            pltpu.make_async_copy(rrows.at[b], acc_sh.at[dx[b]], sr[b]).wait()

        rpt = REL_PAD // NS
        pltpu.sync_copy(rel_hbm.at[pl.ds(s * rpt, rpt)],
                        hrows.at[0, pl.ds(0, rpt)])
        pltpu.sync_copy(hrows.at[0, pl.ds(0, rpt)],
                        rel_sh.at[pl.ds(s * rpt, rpt)])

        @pl.loop(0, ZB)
        def _zr(i):
            @pl.loop(0, RANK // 16)
            def _zc(k):
                hrows[0, i, pl.ds(k * 16, 16)] = jnp.zeros((16,), jnp.float32)

        row0 = s * ROWS_PER_TILE

        @pl.loop(0, ROWS_PER_TILE // ZB)
        def _zcp(k):
            pltpu.sync_copy(hrows.at[0, pl.ds(0, ZB)],
                            acc_sh.at[pl.ds(row0 + k * ZB, ZB)])

        plsc.subcore_barrier()

        ebase = wid * cpt * CHUNK

        @pl.loop(0, cpt, step=2)
        def _pair(g):
            for b in (0, 1):
                j = g + b
                ob = 1 - b
                base = ebase + j * CHUNK

                @pl.when(j >= 2)
                def _():
                    drain_scat(b)

                pltpu.sync_copy(srcx_hbm.at[pl.ds(base, CHUNK)], ci[b])
                pltpu.sync_copy(dstx_hbm.at[pl.ds(base, CHUNK)], dx[b])
                pltpu.sync_copy(etx_hbm.at[pl.ds(base, CHUNK)], ei[b])
                mk_gather_h(b).start()
                mk_gather_r(b).start()

                @pl.when(j >= 1)
                def _():
                    mk_gather_h(ob).wait()
                    mk_gather_r(ob).wait()
                    start_scat(ob)

        mk_gather_h(1).wait()
        mk_gather_r(1).wait()
        start_scat(1)
        drain_scat(0)
        drain_scat(1)

        plsc.subcore_barrier()

        out_base = c * N_PAD + s * ROWS_PER_TILE

        @pl.loop(0, ROWS_PER_TILE // ZB)
        def _wb(k):
            pltpu.sync_copy(acc_sh.at[pl.ds(row0 + k * ZB, ZB)],
                            hrows.at[0, pl.ds(0, ZB)])
            pltpu.sync_copy(hrows.at[0, pl.ds(0, ZB)],
                            acc_hbm.at[pl.ds(out_base + k * ZB, ZB)])

    return body(h, rel_p, srcx, dstx, etx)


def _fin_body(a0_ref, a1_ref, norm_ref, out_ref):
    out_ref[...] = (a0_ref[...] + a1_ref[...]) * norm_ref[...]


def kernel(h, norm, edge_index, edge_type, rel_weight):
    rel = rel_weight[:, :RANK]
    src = edge_index[0].astype(jnp.int32)
    dst = edge_index[1].astype(jnp.int32)
    et = edge_type.astype(jnp.int32)
    n_edges = src.shape[0]

    n_rel = rel.shape[0]
    rel_p = jnp.concatenate(
        [rel, jnp.zeros((REL_PAD - n_rel, RANK), jnp.float32)], axis=0)

    cpt = -(-n_edges // (NW * CHUNK))
    cpt += cpt % 2
    pad = NW * cpt * CHUNK - n_edges
    if pad:
        fill = jnp.arange(pad, dtype=jnp.int32)
        src = jnp.concatenate([src, fill % N_NODES])
        dst = jnp.concatenate([dst, N_NODES + fill % (N_PAD - N_NODES)])
        et = jnp.concatenate([et, n_rel + fill % (REL_PAD - n_rel)])

    acc = _sc_accumulate(h, rel_p, src, dst, et, cpt)

    rows_blk = 80
    out = pl.pallas_call(
        _fin_body,
        grid=(N_NODES // rows_blk,),
        in_specs=[
            pl.BlockSpec((rows_blk, RANK), lambda i: (i, 0)),
            pl.BlockSpec((rows_blk, RANK), lambda i: (i + N_PAD // rows_blk, 0)),
            pl.BlockSpec((rows_blk, 1), lambda i: (i, 0)),
        ],
        out_specs=pl.BlockSpec((rows_blk, RANK), lambda i: (i, 0)),
        out_shape=jax.ShapeDtypeStruct((N_NODES, RANK), jnp.float32),
    )(acc, acc, norm)
    return out


# speedup vs baseline: 1.2424x; 1.2424x over previous
import functools

import jax
import jax.numpy as jnp
from jax import lax
from jax.experimental import pallas as pl
from jax.experimental.pallas import tpu as pltpu
from jax.experimental.pallas import tpu_sc as plsc

N_NODES = 10000
RANK = 128

NC = 2
NS = 16
NW = NC * NS

CHUNK = 120
N_PAD = 10240
ROWS_PER_TILE = N_PAD // NS
ZB = 80
REL_PAD = 256


def _sc_accumulate(h, rel_p, srcx, dstx, etx, cpt):
    mesh = plsc.VectorSubcoreMesh(core_axis_name="c", subcore_axis_name="s")

    @functools.partial(
        pl.kernel,
        mesh=mesh,
        out_type=jax.ShapeDtypeStruct((NC * N_PAD, RANK), jnp.float32),
        scratch_types=[
            pltpu.VMEM_SHARED((N_PAD, RANK), jnp.float32),
            pltpu.VMEM_SHARED((REL_PAD, RANK), jnp.float32),
            [pltpu.VMEM((CHUNK,), jnp.int32)] * 2,
            [pltpu.VMEM((CHUNK,), jnp.int32)] * 2,
            [pltpu.VMEM((CHUNK,), jnp.int32)] * 2,
            pltpu.VMEM((2, CHUNK, RANK), jnp.float32),
            pltpu.VMEM((CHUNK, RANK), jnp.float32),
            [pltpu.SemaphoreType.DMA] * 2,
            pltpu.SemaphoreType.DMA,
            [pltpu.SemaphoreType.DMA] * 2,
            pltpu.SemaphoreType.DMA,
        ],
    )
    def body(h_hbm, rel_hbm, srcx_hbm, dstx_hbm, etx_hbm, acc_hbm,
             acc_sh, rel_sh, ci, dx, ei, hrows, rrows, gh, gr, sh, sr):
        c = lax.axis_index("c")
        s = lax.axis_index("s")
        wid = s * NC + c

        def mk_gather_h(b):
            return pltpu.make_async_copy(h_hbm.at[ci[b]], hrows.at[b], gh[b])

        def mk_gather_r(b):
            return pltpu.make_async_copy(rel_sh.at[ei[b]], rrows, gr)

        def start_scat_h(b):
            pltpu.async_copy(hrows.at[b], acc_sh.at[dx[b]], sh[b], add=True)

        def drain_scat_h(b):
            pltpu.make_async_copy(hrows.at[b], acc_sh.at[dx[b]], sh[b]).wait()

        def start_scat_r(b):
            pltpu.async_copy(rrows, acc_sh.at[dx[b]], sr, add=True)

        def drain_scat_r(b):
            pltpu.make_async_copy(rrows, acc_sh.at[dx[b]], sr).wait()

        rpt = REL_PAD // NS
        pltpu.sync_copy(rel_hbm.at[pl.ds(s * rpt, rpt)],
                        hrows.at[0, pl.ds(0, rpt)])
        pltpu.sync_copy(hrows.at[0, pl.ds(0, rpt)],
                        rel_sh.at[pl.ds(s * rpt, rpt)])

        @pl.loop(0, ZB)
        def _zr(i):
            @pl.loop(0, RANK // 16)
            def _zc(k):
                hrows[0, i, pl.ds(k * 16, 16)] = jnp.zeros((16,), jnp.float32)

        row0 = s * ROWS_PER_TILE

        @pl.loop(0, ROWS_PER_TILE // ZB)
        def _zcp(k):
            pltpu.sync_copy(hrows.at[0, pl.ds(0, ZB)],
                            acc_sh.at[pl.ds(row0 + k * ZB, ZB)])

        plsc.subcore_barrier()

        ebase = wid * cpt * CHUNK

        @pl.loop(0, cpt, step=2)
        def _pair(g):
            for b in (0, 1):
                j = g + b
                ob = 1 - b
                base = ebase + j * CHUNK

                @pl.when(j >= 2)
                def _():
                    drain_scat_h(b)

                pltpu.sync_copy(srcx_hbm.at[pl.ds(base, CHUNK)], ci[b])
                pltpu.sync_copy(dstx_hbm.at[pl.ds(base, CHUNK)], dx[b])
                pltpu.sync_copy(etx_hbm.at[pl.ds(base, CHUNK)], ei[b])
                mk_gather_h(b).start()

                @pl.when(j >= 1)
                def _():
                    mk_gather_h(ob).wait()
                    start_scat_h(ob)

                    @pl.when(j >= 2)
                    def _():
                        drain_scat_r(b)

                    mk_gather_r(ob).start()
                    mk_gather_r(ob).wait()
                    start_scat_r(ob)

        mk_gather_h(1).wait()
        start_scat_h(1)
        drain_scat_r(0)
        mk_gather_r(1).start()
        mk_gather_r(1).wait()
        start_scat_r(1)
        drain_scat_h(0)
        drain_scat_h(1)
        drain_scat_r(1)

        plsc.subcore_barrier()

        out_base = c * N_PAD + s * ROWS_PER_TILE

        @pl.loop(0, ROWS_PER_TILE // ZB)
        def _wb(k):
            pltpu.sync_copy(acc_sh.at[pl.ds(row0 + k * ZB, ZB)],
                            hrows.at[0, pl.ds(0, ZB)])
            pltpu.sync_copy(hrows.at[0, pl.ds(0, ZB)],
                            acc_hbm.at[pl.ds(out_base + k * ZB, ZB)])

    return body(h, rel_p, srcx, dstx, etx)


def _fin_body(a0_ref, a1_ref, norm_ref, out_ref):
    out_ref[...] = (a0_ref[...] + a1_ref[...]) * norm_ref[...]


def kernel(h, norm, edge_index, edge_type, rel_weight):
    rel = rel_weight[:, :RANK]
    src = edge_index[0].astype(jnp.int32)
    dst = edge_index[1].astype(jnp.int32)
    et = edge_type.astype(jnp.int32)
    n_edges = src.shape[0]

    n_rel = rel.shape[0]
    rel_p = jnp.concatenate(
        [rel, jnp.zeros((REL_PAD - n_rel, RANK), jnp.float32)], axis=0)

    cpt = -(-n_edges // (NW * CHUNK))
    cpt += cpt % 2
    pad = NW * cpt * CHUNK - n_edges
    if pad:
        fill = jnp.arange(pad, dtype=jnp.int32)
        src = jnp.concatenate([src, fill % N_NODES])
        dst = jnp.concatenate([dst, N_NODES + fill % (N_PAD - N_NODES)])
        et = jnp.concatenate([et, n_rel + fill % (REL_PAD - n_rel)])

    acc = _sc_accumulate(h, rel_p, src, dst, et, cpt)

    rows_blk = 1000
    out = pl.pallas_call(
        _fin_body,
        grid=(N_NODES // rows_blk,),
        in_specs=[
            pl.BlockSpec((rows_blk, RANK), lambda i: (i, 0)),
            pl.BlockSpec((rows_blk, RANK), lambda i: (i, 0)),
            pl.BlockSpec((rows_blk, 1), lambda i: (i, 0)),
        ],
        out_specs=pl.BlockSpec((rows_blk, RANK), lambda i: (i, 0)),
        out_shape=jax.ShapeDtypeStruct((N_NODES, RANK), jnp.float32),
    )(acc[:N_NODES], acc[N_PAD:N_PAD + N_NODES], norm)
    return out
